# baseline (speedup 1.0000x reference)
import functools

import jax
import jax.numpy as jnp
from jax import lax
from jax.experimental import pallas as pl
from jax.experimental.pallas import tpu as pltpu
from jax.experimental.pallas import tpu_sc as plsc

BATCH = 1024
SEQ = 200
EMB_DIM = 64

NC, NS = 2, 16
NW = NC * NS
TOTAL_ROWS = BATCH * SEQ
ROWS_PER_W = TOTAL_ROWS // NW
CHUNK = 128
NCHUNK = ROWS_PER_W // CHUNK


def _pos_embedding_doubled():
    position = jnp.arange(0, SEQ, dtype=jnp.float32)[:, None]
    div_term = jnp.exp(
        jnp.arange(0, EMB_DIM, 2, dtype=jnp.float32)
        * (-jnp.log(jnp.array(10000.0)) / EMB_DIM)
    )
    pe = jnp.zeros((SEQ, EMB_DIM), dtype=jnp.float32)
    pe = pe.at[:, 0::2].set(jnp.sin(position * div_term))
    pe = pe.at[:, 1::2].set(jnp.cos(position * div_term))
    return jnp.concatenate([pe, pe], axis=0)


def _sc_embed(x2d, pe2, table):
    mesh = plsc.VectorSubcoreMesh(core_axis_name="c", subcore_axis_name="s")

    @functools.partial(
        pl.kernel,
        out_type=jax.ShapeDtypeStruct((TOTAL_ROWS, EMB_DIM), jnp.float32),
        mesh=mesh,
        scratch_types=[
            pltpu.VMEM((ROWS_PER_W,), jnp.int32),
            pltpu.VMEM_SHARED((2 * SEQ, EMB_DIM), jnp.float32),
            pltpu.VMEM((CHUNK, EMB_DIM), jnp.float32),
            pltpu.SemaphoreType.DMA,
        ],
        compiler_params=pltpu.CompilerParams(use_tc_tiling_on_sc=False),
    )
    def k(x_hbm, pe_hbm, table_hbm, out_hbm, idx_v, pe_v, rows_v, sem):
        sid = lax.axis_index("s")
        wid = sid * NC + lax.axis_index("c")
        wbase = wid * ROWS_PER_W
        pltpu.sync_copy(x_hbm.at[pl.ds(wbase, ROWS_PER_W)], idx_v)

        @pl.when(sid == 0)
        def _():
            pltpu.sync_copy(pe_hbm, pe_v)

        plsc.subcore_barrier()

        @pl.loop(0, NCHUNK)
        def _(j):
            row_base = wbase + j * CHUNK
            r0 = lax.rem(row_base, SEQ)
            pltpu.sync_copy(pe_v.at[pl.ds(r0, CHUNK)], rows_v)
            pltpu.async_copy(
                table_hbm.at[idx_v.at[pl.ds(j * CHUNK, CHUNK)]],
                rows_v, sem, add=True,
            ).wait()
            pltpu.sync_copy(rows_v, out_hbm.at[pl.ds(row_base, CHUNK)])

    return k(x2d, pe2, table)


def kernel(x, table):
    pe2 = _pos_embedding_doubled()
    out = _sc_embed(x.reshape(TOTAL_ROWS), pe2, table)
    return out.reshape(BATCH, SEQ, EMB_DIM)


# speedup vs baseline: 1.0640x; 1.0640x over previous
import functools

import jax
import jax.numpy as jnp
from jax import lax
from jax.experimental import pallas as pl
from jax.experimental.pallas import tpu as pltpu
from jax.experimental.pallas import tpu_sc as plsc

BATCH = 1024
SEQ = 200
EMB_DIM = 64

NC, NS = 2, 16
NW = NC * NS
TOTAL_ROWS = BATCH * SEQ
ROWS_PER_W = TOTAL_ROWS // NW
CHUNK = 128
NCHUNK = ROWS_PER_W // CHUNK
NBUF = 5
LEAD = 3


def _pos_embedding_doubled():
    position = jnp.arange(0, SEQ, dtype=jnp.float32)[:, None]
    div_term = jnp.exp(
        jnp.arange(0, EMB_DIM, 2, dtype=jnp.float32)
        * (-jnp.log(jnp.array(10000.0)) / EMB_DIM)
    )
    pe = jnp.zeros((SEQ, EMB_DIM), dtype=jnp.float32)
    pe = pe.at[:, 0::2].set(jnp.sin(position * div_term))
    pe = pe.at[:, 1::2].set(jnp.cos(position * div_term))
    return jnp.concatenate([pe, pe], axis=0)


def _sc_embed(x1d, pe2, table):
    mesh = plsc.VectorSubcoreMesh(core_axis_name="c", subcore_axis_name="s")

    @functools.partial(
        pl.kernel,
        out_type=jax.ShapeDtypeStruct((TOTAL_ROWS, EMB_DIM), jnp.float32),
        mesh=mesh,
        scratch_types=[
            pltpu.VMEM((ROWS_PER_W,), jnp.int32),
            pltpu.VMEM_SHARED((2 * SEQ, EMB_DIM), jnp.float32),
            pltpu.VMEM((NBUF, CHUNK, EMB_DIM), jnp.float32),
            pltpu.SemaphoreType.DMA((NBUF,)),
            pltpu.SemaphoreType.DMA((NBUF,)),
            pltpu.SemaphoreType.DMA,
        ],
        compiler_params=pltpu.CompilerParams(use_tc_tiling_on_sc=False),
    )
    def k(x_hbm, pe_hbm, table_hbm, out_hbm, idx_v, pe_v, rows_v,
          gsem, osem, isem):
        sid = lax.axis_index("s")
        wid = sid * NC + lax.axis_index("c")
        wbase = wid * ROWS_PER_W
        pltpu.async_copy(x_hbm.at[pl.ds(wbase, ROWS_PER_W)], idx_v, isem)

        @pl.when(sid == 0)
        def _():
            pltpu.sync_copy(pe_hbm, pe_v)

        plsc.subcore_barrier()
        pltpu.make_async_copy(
            x_hbm.at[pl.ds(wbase, ROWS_PER_W)], idx_v, isem).wait()

        def fill(b, j):
            r0 = lax.rem(wbase + j * CHUNK, SEQ)
            return pltpu.async_copy(
                pe_v.at[pl.ds(r0, CHUNK)], rows_v.at[b], gsem.at[b])

        def gather(b, j):
            pltpu.async_copy(
                table_hbm.at[idx_v.at[pl.ds(j * CHUNK, CHUNK)]],
                rows_v.at[b], gsem.at[b], add=True)

        def wait_bytes_of(b, sem):
            pltpu.make_async_copy(
                out_hbm.at[pl.ds(0, CHUNK)], rows_v.at[b], sem.at[b]).wait()

        for b in range(LEAD):
            fill(b, b).wait()
            gather(b, b)

        @pl.loop(0, NCHUNK, step=NBUF)
        def _(base):
            for b in range(NBUF):
                j = base + b
                bg = (b + LEAD) % NBUF
                jg = j + LEAD

                @pl.when(jg >= NBUF)
                def _():
                    wait_bytes_of(bg, osem)

                @pl.when(jg < NCHUNK)
                def _():
                    fill(bg, jg).wait()
                    gather(bg, jg)

                wait_bytes_of(b, gsem)
                pltpu.async_copy(
                    rows_v.at[b],
                    out_hbm.at[pl.ds(wbase + j * CHUNK, CHUNK)],
                    osem.at[b])

        for b in range(NBUF):
            if ((b - LEAD) % NBUF) < (NBUF - LEAD):
                wait_bytes_of(b, osem)

    return k(x1d, pe2, table)


def kernel(x, table):
    pe2 = _pos_embedding_doubled()
    out = _sc_embed(x.reshape(TOTAL_ROWS), pe2, table)
    return out.reshape(BATCH, SEQ, EMB_DIM)
